# baseline (speedup 1.0000x reference)
import jax
import jax.numpy as jnp
from jax import lax
from jax.experimental import pallas as pl
from jax.experimental.pallas import tpu as pltpu

NBUF = 8
CHUNK = 32


def _body(top_smem, top_ref, shifted_ref, trees_hbm, hid_hbm,
          out_hbm, head_hbm, inbuf, outbuf, gbuf, in_sems, out_sems, gsem):
    m, s, i = trees_hbm.shape

    n_chunks = m // CHUNK

    def in_copy(c, b):
        return pltpu.make_async_copy(
            trees_hbm.at[pl.ds(c * CHUNK, CHUNK)],
            inbuf.at[pl.ds(b * CHUNK, CHUNK)],
            in_sems.at[b],
        )

    def out_copy(c, b):
        return pltpu.make_async_copy(
            outbuf.at[pl.ds(b * CHUNK, CHUNK)],
            out_hbm.at[pl.ds(c * CHUNK, CHUNK)],
            out_sems.at[b],
        )

    for b in range(NBUF):
        in_copy(b, b).start()

    def step(c, carry):
        b = lax.rem(c, NBUF)

        for k in range(CHUNK):
            j = c * CHUNK + k
            t = top_smem[j]
            pltpu.make_async_copy(
                hid_hbm.at[j, pl.ds(t + 1, 1)],
                gbuf.at[pl.ds(j, 1)],
                gsem,
            ).start()

        @pl.when(c >= NBUF)
        def _():
            out_copy(c - NBUF, b).wait()

        in_copy(c, b).wait()
        rows = inbuf[pl.ds(b * CHUNK, CHUNK)]
        top = top_ref[pl.ds(c * CHUNK, CHUNK)]
        shifted = shifted_ref[pl.ds(c * CHUNK, CHUNK)]
        stack_iota = lax.broadcasted_iota(jnp.int32, (CHUNK, s, i), 1)
        outbuf[pl.ds(b * CHUNK, CHUNK)] = jnp.where(
            stack_iota == top, shifted, rows
        )
        out_copy(c, b).start()

        @pl.when(c + NBUF < n_chunks)
        def _():
            in_copy(c + NBUF, b).start()

        return carry

    lax.fori_loop(0, n_chunks, step, 0)
    for b in range(NBUF):
        c = n_chunks - NBUF + b
        out_copy(c, c % NBUF).wait()
    pltpu.make_async_copy(head_hbm, gbuf, gsem).wait()
    pltpu.sync_copy(gbuf, head_hbm)


def kernel(trees, hiddens, shifted_embs, top_position):
    m, s, i = trees.shape
    h = hiddens.shape[2]
    call = pl.pallas_call(
        _body,
        in_specs=[
            pl.BlockSpec(memory_space=pltpu.SMEM),
            pl.BlockSpec(memory_space=pltpu.VMEM),
            pl.BlockSpec(memory_space=pltpu.VMEM),
            pl.BlockSpec(memory_space=pltpu.HBM),
            pl.BlockSpec(memory_space=pltpu.HBM),
        ],
        out_specs=(
            pl.BlockSpec(memory_space=pltpu.HBM),
            pl.BlockSpec(memory_space=pltpu.HBM),
        ),
        scratch_shapes=[
            pltpu.VMEM((NBUF * CHUNK, s, i), trees.dtype),
            pltpu.VMEM((NBUF * CHUNK, s, i), trees.dtype),
            pltpu.VMEM((m, h), hiddens.dtype),
            pltpu.SemaphoreType.DMA((NBUF,)),
            pltpu.SemaphoreType.DMA((NBUF,)),
            pltpu.SemaphoreType.DMA,
        ],
        out_shape=(
            jax.ShapeDtypeStruct((m, s, i), trees.dtype),
            jax.ShapeDtypeStruct((m, h), hiddens.dtype),
        ),
    )
    new_trees, hidden_head = call(
        top_position,
        top_position.reshape(m, 1, 1),
        shifted_embs.reshape(m, 1, i),
        trees,
        hiddens,
    )
    return (new_trees, hidden_head)


# speedup vs baseline: 1.1166x; 1.1166x over previous
import functools

import jax
import jax.numpy as jnp
from jax import lax
from jax.experimental import pallas as pl
from jax.experimental.pallas import tpu as pltpu
from jax.experimental.pallas import tpu_sc as plsc

NBUF = 8
CHUNK = 32
SC_NBUF = 4
SC_CHUNK = 2
TC_FRAC_NUM, TC_FRAC_DEN = 1, 2


def _creator_body(out_ref, seed_v, sem):
    pltpu.make_async_copy(seed_v, out_ref.at[pl.ds(0, 8)], sem).start()
    pltpu.make_async_copy(seed_v, out_ref.at[pl.ds(0, 8)], sem).wait()


def _make_creator(m, s, i, dtype):
    return pl.pallas_call(
        _creator_body,
        out_specs=pl.BlockSpec(memory_space=pltpu.HBM),
        scratch_shapes=[
            pltpu.VMEM((8, s, i), dtype),
            pltpu.SemaphoreType.DMA,
        ],
        out_shape=jax.ShapeDtypeStruct((m, s, i), dtype),
    )


def _tc_body(top_ref, shifted_ref, trees_hbm, target_hbm, dum_ref,
             inbuf, outbuf, in_sems, out_sems):
    s, i = trees_hbm.shape[1], trees_hbm.shape[2]
    m_tc = (trees_hbm.shape[0] * TC_FRAC_NUM) // TC_FRAC_DEN
    n_chunks = m_tc // CHUNK

    def in_copy(c, b):
        return pltpu.make_async_copy(
            trees_hbm.at[pl.ds(c * CHUNK, CHUNK)],
            inbuf.at[pl.ds(b * CHUNK, CHUNK)],
            in_sems.at[b],
        )

    def out_copy(c, b):
        return pltpu.make_async_copy(
            outbuf.at[pl.ds(b * CHUNK, CHUNK)],
            target_hbm.at[pl.ds(c * CHUNK, CHUNK)],
            out_sems.at[b],
        )

    dum_ref[...] = jnp.zeros_like(dum_ref)
    for b in range(NBUF):
        in_copy(b, b).start()

    def step(c, carry):
        b = lax.rem(c, NBUF)

        @pl.when(c >= NBUF)
        def _():
            out_copy(c - NBUF, b).wait()

        in_copy(c, b).wait()
        rows = inbuf[pl.ds(b * CHUNK, CHUNK)]
        top = top_ref[pl.ds(c * CHUNK, CHUNK)]
        shifted = shifted_ref[pl.ds(c * CHUNK, CHUNK)]
        stack_iota = lax.broadcasted_iota(jnp.int32, (CHUNK, s, i), 1)
        outbuf[pl.ds(b * CHUNK, CHUNK)] = jnp.where(
            stack_iota == top, shifted, rows
        )
        out_copy(c, b).start()

        @pl.when(c + NBUF < n_chunks)
        def _():
            in_copy(c + NBUF, b).start()

        return carry

    lax.fori_loop(0, n_chunks, step, 0)
    for b in range(NBUF):
        c = n_chunks - NBUF + b
        out_copy(c, c % NBUF).wait()


def _make_tc_call(m, s, i, dtype):
    m_tc = (m * TC_FRAC_NUM) // TC_FRAC_DEN
    return pl.pallas_call(
        _tc_body,
        in_specs=[
            pl.BlockSpec(memory_space=pltpu.VMEM),
            pl.BlockSpec(memory_space=pltpu.VMEM),
            pl.BlockSpec(memory_space=pltpu.HBM),
            pl.BlockSpec(memory_space=pltpu.HBM),
        ],
        out_specs=pl.BlockSpec(memory_space=pltpu.VMEM),
        scratch_shapes=[
            pltpu.VMEM((NBUF * CHUNK, s, i), dtype),
            pltpu.VMEM((NBUF * CHUNK, s, i), dtype),
            pltpu.SemaphoreType.DMA((NBUF,)),
            pltpu.SemaphoreType.DMA((NBUF,)),
        ],
        out_shape=jax.ShapeDtypeStruct((8, 128), dtype),
    )


def _make_sc_call(m, s, i, slots, h, dtype):
    info = plsc.get_sparse_core_info()
    nw = info.num_cores * info.num_subcores
    m_sc = m - (m * TC_FRAC_NUM) // TC_FRAC_DEN
    sc_base0 = m - m_sc
    rows_w = m_sc // nw
    g_per_w = m // nw
    n_chunks = rows_w // SC_CHUNK
    mesh = plsc.VectorSubcoreMesh(core_axis_name="c", subcore_axis_name="s")

    @functools.partial(
        pl.kernel,
        mesh=mesh,
        out_type=jax.ShapeDtypeStruct((m, h), dtype),
        scratch_types=[
            pltpu.VMEM((g_per_w,), jnp.int32),
            pltpu.VMEM((rows_w,), jnp.int32),
            pltpu.VMEM((g_per_w, h), dtype),
            pltpu.VMEM((rows_w, h), dtype),
            [pltpu.VMEM((SC_CHUNK, s, i), dtype) for _ in range(SC_NBUF)],
            [pltpu.SemaphoreType.DMA for _ in range(SC_NBUF)],
            [pltpu.SemaphoreType.DMA for _ in range(SC_NBUF)],
            pltpu.SemaphoreType.DMA,
            pltpu.SemaphoreType.DMA,
        ],
    )
    def sc_k(top_hbm, trees_hbm, shifted_hbm, hid_hbm, target_hbm, head_hbm,
             top_v, top_sv, rows_v, shifted_v, bufs, in_sems, out_sems,
             sem_g, sem_s):
        wid = lax.axis_index("s") * info.num_cores + lax.axis_index("c")
        gbase = wid * g_per_w
        gslab = pl.ds(gbase, g_per_w)
        base = sc_base0 + wid * rows_w

        def chunk(c):
            return pl.ds(base + c * SC_CHUNK, SC_CHUNK)

        pltpu.sync_copy(top_hbm.at[gslab], top_v)
        pltpu.sync_copy(top_hbm.at[pl.ds(base, rows_w)], top_sv)
        pltpu.sync_copy(shifted_hbm.at[pl.ds(base, rows_w)], shifted_v)

        for c in range(g_per_w // 16):
            tv = top_v[pl.ds(c * 16, 16)]
            for k in range(16):
                j = c * 16 + k
                pltpu.make_async_copy(
                    hid_hbm.at[gbase + j, pl.ds(tv[k] + 1, 1)],
                    rows_v.at[pl.ds(j, 1)],
                    sem_g,
                ).start()

        for b in range(SC_NBUF):
            pltpu.make_async_copy(trees_hbm.at[chunk(b)], bufs[b], in_sems[b]).start()
        for c in range(n_chunks):
            b = c % SC_NBUF
            pltpu.make_async_copy(trees_hbm.at[chunk(c)], bufs[b], in_sems[b]).wait()
            pltpu.make_async_copy(bufs[b], target_hbm.at[chunk(c)], out_sems[b]).start()
            nxt = c + SC_NBUF
            if nxt < n_chunks:
                pltpu.make_async_copy(bufs[b], target_hbm.at[chunk(c)], out_sems[b]).wait()
                pltpu.make_async_copy(trees_hbm.at[chunk(nxt)], bufs[b], in_sems[b]).start()
        for c in range(max(n_chunks - SC_NBUF, 0), n_chunks):
            b = c % SC_NBUF
            pltpu.make_async_copy(bufs[b], target_hbm.at[chunk(c)], out_sems[b]).wait()

        for c in range(rows_w // 16):
            tv = top_sv[pl.ds(c * 16, 16)]
            for k in range(16):
                j = c * 16 + k
                pltpu.make_async_copy(
                    shifted_v.at[pl.ds(j, 1)],
                    target_hbm.at[base + j, pl.ds(tv[k], 1)],
                    sem_s,
                ).start()

        pltpu.make_async_copy(head_hbm.at[gslab], rows_v, sem_g).wait()
        pltpu.sync_copy(rows_v, head_hbm.at[gslab])
        pltpu.make_async_copy(head_hbm.at[pl.ds(gbase, rows_w)], shifted_v, sem_s).wait()

    return sc_k


def kernel(trees, hiddens, shifted_embs, top_position):
    m, s, i = trees.shape
    slots = hiddens.shape[1]
    h = hiddens.shape[2]
    m_tc = (m * TC_FRAC_NUM) // TC_FRAC_DEN
    target = _make_creator(m, s, i, trees.dtype)()
    tc_dummy = _make_tc_call(m, s, i, trees.dtype)(
        top_position[:m_tc].reshape(m_tc, 1, 1),
        shifted_embs[:m_tc].reshape(m_tc, 1, i),
        trees,
        target,
    )
    hidden_head = _make_sc_call(m, s, i, slots, h, trees.dtype)(
        top_position, trees, shifted_embs, hiddens, target
    )
    new_trees, _, hidden_head = lax.optimization_barrier(
        (target, tc_dummy, hidden_head)
    )
    return (new_trees, hidden_head)


# speedup vs baseline: 1.1916x; 1.0672x over previous
import functools

import jax
import jax.numpy as jnp
from jax import lax
from jax.experimental import pallas as pl
from jax.experimental.pallas import tpu as pltpu
from jax.experimental.pallas import tpu_sc as plsc

NBUF = 8
CHUNK = 32
SC_NBUF = 4
SC_CHUNK = 2
TC_FRAC_NUM, TC_FRAC_DEN = 5, 8


def _creator_body(out_ref, seed_v, sem):
    pltpu.make_async_copy(seed_v, out_ref.at[pl.ds(0, 8)], sem).start()
    pltpu.make_async_copy(seed_v, out_ref.at[pl.ds(0, 8)], sem).wait()


def _make_creator(m, s, i, dtype):
    return pl.pallas_call(
        _creator_body,
        out_specs=pl.BlockSpec(memory_space=pltpu.HBM),
        scratch_shapes=[
            pltpu.VMEM((8, s, i), dtype),
            pltpu.SemaphoreType.DMA,
        ],
        out_shape=jax.ShapeDtypeStruct((m, s, i), dtype),
    )


def _tc_body(top_ref, shifted_ref, trees_hbm, target_hbm, dum_ref,
             inbuf, outbuf, in_sems, out_sems):
    s, i = trees_hbm.shape[1], trees_hbm.shape[2]
    m_tc = (trees_hbm.shape[0] * TC_FRAC_NUM) // TC_FRAC_DEN
    n_chunks = m_tc // CHUNK

    def in_copy(c, b):
        return pltpu.make_async_copy(
            trees_hbm.at[pl.ds(c * CHUNK, CHUNK)],
            inbuf.at[pl.ds(b * CHUNK, CHUNK)],
            in_sems.at[b],
        )

    def out_copy(c, b):
        return pltpu.make_async_copy(
            outbuf.at[pl.ds(b * CHUNK, CHUNK)],
            target_hbm.at[pl.ds(c * CHUNK, CHUNK)],
            out_sems.at[b],
        )

    dum_ref[...] = jnp.zeros_like(dum_ref)
    for b in range(NBUF):
        in_copy(b, b).start()

    def step(c, carry):
        b = lax.rem(c, NBUF)

        @pl.when(c >= NBUF)
        def _():
            out_copy(c - NBUF, b).wait()

        in_copy(c, b).wait()
        rows = inbuf[pl.ds(b * CHUNK, CHUNK)]
        top = top_ref[pl.ds(c * CHUNK, CHUNK)]
        shifted = shifted_ref[pl.ds(c * CHUNK, CHUNK)]
        stack_iota = lax.broadcasted_iota(jnp.int32, (CHUNK, s, i), 1)
        outbuf[pl.ds(b * CHUNK, CHUNK)] = jnp.where(
            stack_iota == top, shifted, rows
        )
        out_copy(c, b).start()

        @pl.when(c + NBUF < n_chunks)
        def _():
            in_copy(c + NBUF, b).start()

        return carry

    lax.fori_loop(0, n_chunks, step, 0)
    for b in range(NBUF):
        c = n_chunks - NBUF + b
        out_copy(c, c % NBUF).wait()


def _make_tc_call(m, s, i, dtype):
    m_tc = (m * TC_FRAC_NUM) // TC_FRAC_DEN
    return pl.pallas_call(
        _tc_body,
        in_specs=[
            pl.BlockSpec(memory_space=pltpu.VMEM),
            pl.BlockSpec(memory_space=pltpu.VMEM),
            pl.BlockSpec(memory_space=pltpu.HBM),
            pl.BlockSpec(memory_space=pltpu.HBM),
        ],
        out_specs=pl.BlockSpec(memory_space=pltpu.VMEM),
        scratch_shapes=[
            pltpu.VMEM((NBUF * CHUNK, s, i), dtype),
            pltpu.VMEM((NBUF * CHUNK, s, i), dtype),
            pltpu.SemaphoreType.DMA((NBUF,)),
            pltpu.SemaphoreType.DMA((NBUF,)),
        ],
        out_shape=jax.ShapeDtypeStruct((8, 128), dtype),
    )


def _make_sc_call(m, s, i, slots, h, dtype):
    info = plsc.get_sparse_core_info()
    nw = info.num_cores * info.num_subcores
    m_sc = m - (m * TC_FRAC_NUM) // TC_FRAC_DEN
    sc_base0 = m - m_sc
    rows_w = m_sc // nw
    g_per_w = m // nw
    n_chunks = rows_w // SC_CHUNK
    mesh = plsc.VectorSubcoreMesh(core_axis_name="c", subcore_axis_name="s")

    @functools.partial(
        pl.kernel,
        mesh=mesh,
        out_type=jax.ShapeDtypeStruct((m, h), dtype),
        scratch_types=[
            pltpu.VMEM((g_per_w,), jnp.int32),
            pltpu.VMEM((rows_w,), jnp.int32),
            pltpu.VMEM((g_per_w, h), dtype),
            pltpu.VMEM((rows_w, h), dtype),
            [pltpu.VMEM((SC_CHUNK, s, i), dtype) for _ in range(SC_NBUF)],
            [pltpu.SemaphoreType.DMA for _ in range(SC_NBUF)],
            [pltpu.SemaphoreType.DMA for _ in range(SC_NBUF)],
            pltpu.SemaphoreType.DMA,
            pltpu.SemaphoreType.DMA,
        ],
    )
    def sc_k(top_hbm, trees_hbm, shifted_hbm, hid_hbm, target_hbm, head_hbm,
             top_v, top_sv, rows_v, shifted_v, bufs, in_sems, out_sems,
             sem_g, sem_s):
        wid = lax.axis_index("s") * info.num_cores + lax.axis_index("c")
        gbase = wid * g_per_w
        gslab = pl.ds(gbase, g_per_w)
        base = sc_base0 + wid * rows_w

        def chunk(c):
            return pl.ds(base + c * SC_CHUNK, SC_CHUNK)

        pltpu.sync_copy(top_hbm.at[gslab], top_v)
        pltpu.sync_copy(top_hbm.at[pl.ds(base, rows_w)], top_sv)
        pltpu.sync_copy(shifted_hbm.at[pl.ds(base, rows_w)], shifted_v)

        for c in range(g_per_w // 16):
            tv = top_v[pl.ds(c * 16, 16)]
            for k in range(16):
                j = c * 16 + k
                pltpu.make_async_copy(
                    hid_hbm.at[gbase + j, pl.ds(tv[k] + 1, 1)],
                    rows_v.at[pl.ds(j, 1)],
                    sem_g,
                ).start()

        for b in range(SC_NBUF):
            pltpu.make_async_copy(trees_hbm.at[chunk(b)], bufs[b], in_sems[b]).start()
        for c in range(n_chunks):
            b = c % SC_NBUF
            pltpu.make_async_copy(trees_hbm.at[chunk(c)], bufs[b], in_sems[b]).wait()
            pltpu.make_async_copy(bufs[b], target_hbm.at[chunk(c)], out_sems[b]).start()
            nxt = c + SC_NBUF
            if nxt < n_chunks:
                pltpu.make_async_copy(bufs[b], target_hbm.at[chunk(c)], out_sems[b]).wait()
                pltpu.make_async_copy(trees_hbm.at[chunk(nxt)], bufs[b], in_sems[b]).start()
        for c in range(max(n_chunks - SC_NBUF, 0), n_chunks):
            b = c % SC_NBUF
            pltpu.make_async_copy(bufs[b], target_hbm.at[chunk(c)], out_sems[b]).wait()

        for c in range(rows_w // 16):
            tv = top_sv[pl.ds(c * 16, 16)]
            for k in range(16):
                j = c * 16 + k
                pltpu.make_async_copy(
                    shifted_v.at[pl.ds(j, 1)],
                    target_hbm.at[base + j, pl.ds(tv[k], 1)],
                    sem_s,
                ).start()

        pltpu.make_async_copy(head_hbm.at[gslab], rows_v, sem_g).wait()
        pltpu.sync_copy(rows_v, head_hbm.at[gslab])
        pltpu.make_async_copy(head_hbm.at[pl.ds(gbase, rows_w)], shifted_v, sem_s).wait()

    return sc_k


def kernel(trees, hiddens, shifted_embs, top_position):
    m, s, i = trees.shape
    slots = hiddens.shape[1]
    h = hiddens.shape[2]
    m_tc = (m * TC_FRAC_NUM) // TC_FRAC_DEN
    target = _make_creator(m, s, i, trees.dtype)()
    tc_dummy = _make_tc_call(m, s, i, trees.dtype)(
        top_position[:m_tc].reshape(m_tc, 1, 1),
        shifted_embs[:m_tc].reshape(m_tc, 1, i),
        trees,
        target,
    )
    hidden_head = _make_sc_call(m, s, i, slots, h, trees.dtype)(
        top_position, trees, shifted_embs, hiddens, target
    )
    new_trees, _, hidden_head = lax.optimization_barrier(
        (target, tc_dummy, hidden_head)
    )
    return (new_trees, hidden_head)


# speedup vs baseline: 1.2742x; 1.0693x over previous
import functools

import jax
import jax.numpy as jnp
from jax import lax
from jax.experimental import pallas as pl
from jax.experimental.pallas import tpu as pltpu
from jax.experimental.pallas import tpu_sc as plsc

NBUF = 8
CHUNK = 32
SC_NBUF = 4
SC_CHUNK = 2
TC_FRAC_NUM, TC_FRAC_DEN = 3, 4


def _creator_body(out_ref, seed_v, sem):
    pltpu.make_async_copy(seed_v, out_ref.at[pl.ds(0, 8)], sem).start()
    pltpu.make_async_copy(seed_v, out_ref.at[pl.ds(0, 8)], sem).wait()


def _make_creator(m, s, i, dtype):
    return pl.pallas_call(
        _creator_body,
        out_specs=pl.BlockSpec(memory_space=pltpu.HBM),
        scratch_shapes=[
            pltpu.VMEM((8, s, i), dtype),
            pltpu.SemaphoreType.DMA,
        ],
        out_shape=jax.ShapeDtypeStruct((m, s, i), dtype),
    )


def _tc_body(top_ref, shifted_ref, trees_hbm, target_hbm, dum_ref,
             inbuf, outbuf, in_sems, out_sems):
    s, i = trees_hbm.shape[1], trees_hbm.shape[2]
    m_tc = (trees_hbm.shape[0] * TC_FRAC_NUM) // TC_FRAC_DEN
    n_chunks = m_tc // CHUNK

    def in_copy(c, b):
        return pltpu.make_async_copy(
            trees_hbm.at[pl.ds(c * CHUNK, CHUNK)],
            inbuf.at[pl.ds(b * CHUNK, CHUNK)],
            in_sems.at[b],
        )

    def out_copy(c, b):
        return pltpu.make_async_copy(
            outbuf.at[pl.ds(b * CHUNK, CHUNK)],
            target_hbm.at[pl.ds(c * CHUNK, CHUNK)],
            out_sems.at[b],
        )

    dum_ref[...] = jnp.zeros_like(dum_ref)
    for b in range(NBUF):
        in_copy(b, b).start()

    def step(c, carry):
        b = lax.rem(c, NBUF)

        @pl.when(c >= NBUF)
        def _():
            out_copy(c - NBUF, b).wait()

        in_copy(c, b).wait()
        rows = inbuf[pl.ds(b * CHUNK, CHUNK)]
        top = top_ref[pl.ds(c * CHUNK, CHUNK)]
        shifted = shifted_ref[pl.ds(c * CHUNK, CHUNK)]
        stack_iota = lax.broadcasted_iota(jnp.int32, (CHUNK, s, i), 1)
        outbuf[pl.ds(b * CHUNK, CHUNK)] = jnp.where(
            stack_iota == top, shifted, rows
        )
        out_copy(c, b).start()

        @pl.when(c + NBUF < n_chunks)
        def _():
            in_copy(c + NBUF, b).start()

        return carry

    lax.fori_loop(0, n_chunks, step, 0)
    for b in range(NBUF):
        c = n_chunks - NBUF + b
        out_copy(c, c % NBUF).wait()


def _make_tc_call(m, s, i, dtype):
    m_tc = (m * TC_FRAC_NUM) // TC_FRAC_DEN
    return pl.pallas_call(
        _tc_body,
        in_specs=[
            pl.BlockSpec(memory_space=pltpu.VMEM),
            pl.BlockSpec(memory_space=pltpu.VMEM),
            pl.BlockSpec(memory_space=pltpu.HBM),
            pl.BlockSpec(memory_space=pltpu.HBM),
        ],
        out_specs=pl.BlockSpec(memory_space=pltpu.VMEM),
        scratch_shapes=[
            pltpu.VMEM((NBUF * CHUNK, s, i), dtype),
            pltpu.VMEM((NBUF * CHUNK, s, i), dtype),
            pltpu.SemaphoreType.DMA((NBUF,)),
            pltpu.SemaphoreType.DMA((NBUF,)),
        ],
        out_shape=jax.ShapeDtypeStruct((8, 128), dtype),
    )


def _make_sc_call(m, s, i, slots, h, dtype):
    info = plsc.get_sparse_core_info()
    nw = info.num_cores * info.num_subcores
    m_sc = m - (m * TC_FRAC_NUM) // TC_FRAC_DEN
    sc_base0 = m - m_sc
    rows_w = m_sc // nw
    g_per_w = m // nw
    n_chunks = rows_w // SC_CHUNK
    mesh = plsc.VectorSubcoreMesh(core_axis_name="c", subcore_axis_name="s")

    @functools.partial(
        pl.kernel,
        mesh=mesh,
        out_type=jax.ShapeDtypeStruct((m, h), dtype),
        scratch_types=[
            pltpu.VMEM((g_per_w,), jnp.int32),
            pltpu.VMEM((rows_w,), jnp.int32),
            pltpu.VMEM((g_per_w, h), dtype),
            pltpu.VMEM((rows_w, h), dtype),
            [pltpu.VMEM((SC_CHUNK, s, i), dtype) for _ in range(SC_NBUF)],
            [pltpu.SemaphoreType.DMA for _ in range(SC_NBUF)],
            [pltpu.SemaphoreType.DMA for _ in range(SC_NBUF)],
            pltpu.SemaphoreType.DMA,
            pltpu.SemaphoreType.DMA,
        ],
    )
    def sc_k(top_hbm, trees_hbm, shifted_hbm, hid_hbm, target_hbm, head_hbm,
             top_v, top_sv, rows_v, shifted_v, bufs, in_sems, out_sems,
             sem_g, sem_s):
        wid = lax.axis_index("s") * info.num_cores + lax.axis_index("c")
        gbase = wid * g_per_w
        gslab = pl.ds(gbase, g_per_w)
        base = sc_base0 + wid * rows_w

        def chunk(c):
            return pl.ds(base + c * SC_CHUNK, SC_CHUNK)

        pltpu.sync_copy(top_hbm.at[gslab], top_v)
        pltpu.sync_copy(top_hbm.at[pl.ds(base, rows_w)], top_sv)
        pltpu.sync_copy(shifted_hbm.at[pl.ds(base, rows_w)], shifted_v)

        for c in range(g_per_w // 16):
            tv = top_v[pl.ds(c * 16, 16)]
            for k in range(16):
                j = c * 16 + k
                pltpu.make_async_copy(
                    hid_hbm.at[gbase + j, pl.ds(tv[k] + 1, 1)],
                    rows_v.at[pl.ds(j, 1)],
                    sem_g,
                ).start()

        for b in range(SC_NBUF):
            pltpu.make_async_copy(trees_hbm.at[chunk(b)], bufs[b], in_sems[b]).start()
        for c in range(n_chunks):
            b = c % SC_NBUF
            pltpu.make_async_copy(trees_hbm.at[chunk(c)], bufs[b], in_sems[b]).wait()
            pltpu.make_async_copy(bufs[b], target_hbm.at[chunk(c)], out_sems[b]).start()
            nxt = c + SC_NBUF
            if nxt < n_chunks:
                pltpu.make_async_copy(bufs[b], target_hbm.at[chunk(c)], out_sems[b]).wait()
                pltpu.make_async_copy(trees_hbm.at[chunk(nxt)], bufs[b], in_sems[b]).start()
        for c in range(max(n_chunks - SC_NBUF, 0), n_chunks):
            b = c % SC_NBUF
            pltpu.make_async_copy(bufs[b], target_hbm.at[chunk(c)], out_sems[b]).wait()

        for c in range(rows_w // 16):
            tv = top_sv[pl.ds(c * 16, 16)]
            for k in range(16):
                j = c * 16 + k
                pltpu.make_async_copy(
                    shifted_v.at[pl.ds(j, 1)],
                    target_hbm.at[base + j, pl.ds(tv[k], 1)],
                    sem_s,
                ).start()

        pltpu.make_async_copy(head_hbm.at[gslab], rows_v, sem_g).wait()
        pltpu.sync_copy(rows_v, head_hbm.at[gslab])
        pltpu.make_async_copy(head_hbm.at[pl.ds(gbase, rows_w)], shifted_v, sem_s).wait()

    return sc_k


def kernel(trees, hiddens, shifted_embs, top_position):
    m, s, i = trees.shape
    slots = hiddens.shape[1]
    h = hiddens.shape[2]
    m_tc = (m * TC_FRAC_NUM) // TC_FRAC_DEN
    target = _make_creator(m, s, i, trees.dtype)()
    tc_dummy = _make_tc_call(m, s, i, trees.dtype)(
        top_position[:m_tc].reshape(m_tc, 1, 1),
        shifted_embs[:m_tc].reshape(m_tc, 1, i),
        trees,
        target,
    )
    hidden_head = _make_sc_call(m, s, i, slots, h, trees.dtype)(
        top_position, trees, shifted_embs, hiddens, target
    )
    new_trees, _, hidden_head = lax.optimization_barrier(
        (target, tc_dummy, hidden_head)
    )
    return (new_trees, hidden_head)


# speedup vs baseline: 1.3636x; 1.0702x over previous
import functools

import jax
import jax.numpy as jnp
from jax import lax
from jax.experimental import pallas as pl
from jax.experimental.pallas import tpu as pltpu
from jax.experimental.pallas import tpu_sc as plsc

NBUF = 8
CHUNK = 32
SC_NBUF = 4
SC_CHUNK = 2
TC_FRAC_NUM, TC_FRAC_DEN = 7, 8


def _creator_body(out_ref, seed_v, sem):
    pltpu.make_async_copy(seed_v, out_ref.at[pl.ds(0, 8)], sem).start()
    pltpu.make_async_copy(seed_v, out_ref.at[pl.ds(0, 8)], sem).wait()


def _make_creator(m, s, i, dtype):
    return pl.pallas_call(
        _creator_body,
        out_specs=pl.BlockSpec(memory_space=pltpu.HBM),
        scratch_shapes=[
            pltpu.VMEM((8, s, i), dtype),
            pltpu.SemaphoreType.DMA,
        ],
        out_shape=jax.ShapeDtypeStruct((m, s, i), dtype),
    )


def _tc_body(top_ref, shifted_ref, trees_hbm, target_hbm, dum_ref,
             inbuf, outbuf, in_sems, out_sems):
    s, i = trees_hbm.shape[1], trees_hbm.shape[2]
    m_tc = (trees_hbm.shape[0] * TC_FRAC_NUM) // TC_FRAC_DEN
    n_chunks = m_tc // CHUNK

    def in_copy(c, b):
        return pltpu.make_async_copy(
            trees_hbm.at[pl.ds(c * CHUNK, CHUNK)],
            inbuf.at[pl.ds(b * CHUNK, CHUNK)],
            in_sems.at[b],
        )

    def out_copy(c, b):
        return pltpu.make_async_copy(
            outbuf.at[pl.ds(b * CHUNK, CHUNK)],
            target_hbm.at[pl.ds(c * CHUNK, CHUNK)],
            out_sems.at[b],
        )

    dum_ref[...] = jnp.zeros_like(dum_ref)
    for b in range(NBUF):
        in_copy(b, b).start()

    def step(c, carry):
        b = lax.rem(c, NBUF)

        @pl.when(c >= NBUF)
        def _():
            out_copy(c - NBUF, b).wait()

        in_copy(c, b).wait()
        rows = inbuf[pl.ds(b * CHUNK, CHUNK)]
        top = top_ref[pl.ds(c * CHUNK, CHUNK)]
        shifted = shifted_ref[pl.ds(c * CHUNK, CHUNK)]
        stack_iota = lax.broadcasted_iota(jnp.int32, (CHUNK, s, i), 1)
        outbuf[pl.ds(b * CHUNK, CHUNK)] = jnp.where(
            stack_iota == top, shifted, rows
        )
        out_copy(c, b).start()

        @pl.when(c + NBUF < n_chunks)
        def _():
            in_copy(c + NBUF, b).start()

        return carry

    lax.fori_loop(0, n_chunks, step, 0)
    for b in range(NBUF):
        c = n_chunks - NBUF + b
        out_copy(c, c % NBUF).wait()


def _make_tc_call(m, s, i, dtype):
    m_tc = (m * TC_FRAC_NUM) // TC_FRAC_DEN
    return pl.pallas_call(
        _tc_body,
        in_specs=[
            pl.BlockSpec(memory_space=pltpu.VMEM),
            pl.BlockSpec(memory_space=pltpu.VMEM),
            pl.BlockSpec(memory_space=pltpu.HBM),
            pl.BlockSpec(memory_space=pltpu.HBM),
        ],
        out_specs=pl.BlockSpec(memory_space=pltpu.VMEM),
        scratch_shapes=[
            pltpu.VMEM((NBUF * CHUNK, s, i), dtype),
            pltpu.VMEM((NBUF * CHUNK, s, i), dtype),
            pltpu.SemaphoreType.DMA((NBUF,)),
            pltpu.SemaphoreType.DMA((NBUF,)),
        ],
        out_shape=jax.ShapeDtypeStruct((8, 128), dtype),
    )


def _make_sc_call(m, s, i, slots, h, dtype):
    info = plsc.get_sparse_core_info()
    nw = info.num_cores * info.num_subcores
    m_sc = m - (m * TC_FRAC_NUM) // TC_FRAC_DEN
    sc_base0 = m - m_sc
    rows_w = m_sc // nw
    g_per_w = m // nw
    n_chunks = rows_w // SC_CHUNK
    mesh = plsc.VectorSubcoreMesh(core_axis_name="c", subcore_axis_name="s")

    @functools.partial(
        pl.kernel,
        mesh=mesh,
        out_type=jax.ShapeDtypeStruct((m, h), dtype),
        scratch_types=[
            pltpu.VMEM((g_per_w,), jnp.int32),
            pltpu.VMEM((rows_w,), jnp.int32),
            pltpu.VMEM((g_per_w, h), dtype),
            pltpu.VMEM((rows_w, h), dtype),
            [pltpu.VMEM((SC_CHUNK, s, i), dtype) for _ in range(SC_NBUF)],
            [pltpu.SemaphoreType.DMA for _ in range(SC_NBUF)],
            [pltpu.SemaphoreType.DMA for _ in range(SC_NBUF)],
            pltpu.SemaphoreType.DMA,
            pltpu.SemaphoreType.DMA,
        ],
    )
    def sc_k(top_hbm, trees_hbm, shifted_hbm, hid_hbm, target_hbm, head_hbm,
             top_v, top_sv, rows_v, shifted_v, bufs, in_sems, out_sems,
             sem_g, sem_s):
        wid = lax.axis_index("s") * info.num_cores + lax.axis_index("c")
        gbase = wid * g_per_w
        gslab = pl.ds(gbase, g_per_w)
        base = sc_base0 + wid * rows_w

        def chunk(c):
            return pl.ds(base + c * SC_CHUNK, SC_CHUNK)

        pltpu.sync_copy(top_hbm.at[gslab], top_v)
        pltpu.sync_copy(top_hbm.at[pl.ds(base, rows_w)], top_sv)
        pltpu.sync_copy(shifted_hbm.at[pl.ds(base, rows_w)], shifted_v)

        for c in range(g_per_w // 16):
            tv = top_v[pl.ds(c * 16, 16)]
            for k in range(16):
                j = c * 16 + k
                pltpu.make_async_copy(
                    hid_hbm.at[gbase + j, pl.ds(tv[k] + 1, 1)],
                    rows_v.at[pl.ds(j, 1)],
                    sem_g,
                ).start()

        for b in range(SC_NBUF):
            pltpu.make_async_copy(trees_hbm.at[chunk(b)], bufs[b], in_sems[b]).start()
        for c in range(n_chunks):
            b = c % SC_NBUF
            pltpu.make_async_copy(trees_hbm.at[chunk(c)], bufs[b], in_sems[b]).wait()
            pltpu.make_async_copy(bufs[b], target_hbm.at[chunk(c)], out_sems[b]).start()
            nxt = c + SC_NBUF
            if nxt < n_chunks:
                pltpu.make_async_copy(bufs[b], target_hbm.at[chunk(c)], out_sems[b]).wait()
                pltpu.make_async_copy(trees_hbm.at[chunk(nxt)], bufs[b], in_sems[b]).start()
        for c in range(max(n_chunks - SC_NBUF, 0), n_chunks):
            b = c % SC_NBUF
            pltpu.make_async_copy(bufs[b], target_hbm.at[chunk(c)], out_sems[b]).wait()

        for c in range(rows_w // 16):
            tv = top_sv[pl.ds(c * 16, 16)]
            for k in range(16):
                j = c * 16 + k
                pltpu.make_async_copy(
                    shifted_v.at[pl.ds(j, 1)],
                    target_hbm.at[base + j, pl.ds(tv[k], 1)],
                    sem_s,
                ).start()

        pltpu.make_async_copy(head_hbm.at[gslab], rows_v, sem_g).wait()
        pltpu.sync_copy(rows_v, head_hbm.at[gslab])
        pltpu.make_async_copy(head_hbm.at[pl.ds(gbase, rows_w)], shifted_v, sem_s).wait()

    return sc_k


def kernel(trees, hiddens, shifted_embs, top_position):
    m, s, i = trees.shape
    slots = hiddens.shape[1]
    h = hiddens.shape[2]
    m_tc = (m * TC_FRAC_NUM) // TC_FRAC_DEN
    target = _make_creator(m, s, i, trees.dtype)()
    tc_dummy = _make_tc_call(m, s, i, trees.dtype)(
        top_position[:m_tc].reshape(m_tc, 1, 1),
        shifted_embs[:m_tc].reshape(m_tc, 1, i),
        trees,
        target,
    )
    hidden_head = _make_sc_call(m, s, i, slots, h, trees.dtype)(
        top_position, trees, shifted_embs, hiddens, target
    )
    new_trees, _, hidden_head = lax.optimization_barrier(
        (target, tc_dummy, hidden_head)
    )
    return (new_trees, hidden_head)
